# baseline (speedup 1.0000x reference)
import functools

import jax
import jax.numpy as jnp
from jax import lax
from jax.experimental import pallas as pl
from jax.experimental.pallas import tpu as pltpu
from jax.experimental.pallas import tpu_sc as plsc

N = 10000
D = 128
E = 320000
NC = 2
NS = 16
NW = NC * NS

EPT = E // NW
CHUNK = 40
CHUNKS = EPT // CHUNK
NP = 10240
RPT = NP // NS
ZCOPIES = RPT // CHUNK

NBUF = 5
NGROUPS = CHUNKS // NBUF


def _segsum_body(x_hbm, src_hbm, dst_hbm, out_hbm, srcf, dstf, acc, *rest):
    bufs = rest[0:NBUF]
    dvs = rest[NBUF:2 * NBUF]
    gsems = rest[2 * NBUF:3 * NBUF]
    ssems = rest[3 * NBUF:4 * NBUF]
    cid = lax.axis_index("c")
    sid = lax.axis_index("s")
    wid = cid * NS + sid

    pltpu.async_copy(src_hbm.at[pl.ds(wid * EPT, EPT)], srcf, gsems[0])
    pltpu.async_copy(dst_hbm.at[pl.ds(wid * EPT, EPT)], dstf, gsems[1])

    zv = jnp.zeros((16,), jnp.float32)

    def _zrow(i, carry):
        for j in range(D // 16):
            bufs[0][i, pl.ds(j * 16, 16)] = zv
        return carry

    lax.fori_loop(0, CHUNK, _zrow, 0)
    for t in range(ZCOPIES):
        pltpu.async_copy(bufs[0], acc.at[pl.ds(sid * RPT + t * CHUNK, CHUNK)],
                         ssems[t % NBUF])
    pltpu.make_async_copy(src_hbm.at[pl.ds(wid * EPT, EPT)], srcf, gsems[0]).wait()
    pltpu.make_async_copy(dst_hbm.at[pl.ds(wid * EPT, EPT)], dstf, gsems[1]).wait()
    for t in range(ZCOPIES):
        pltpu.make_async_copy(bufs[0], acc.at[pl.ds(sid * RPT + t * CHUNK, CHUNK)],
                              ssems[t % NBUF]).wait()
    plsc.subcore_barrier()

    def _copy_dst_idx(j, dv):
        for k in range(CHUNK // 16):
            dv[pl.ds(k * 16, 16)] = dstf[pl.ds(j * CHUNK + k * 16, 16)]
        if CHUNK % 16:
            o = CHUNK - 16
            dv[pl.ds(o, 16)] = dstf[pl.ds(j * CHUNK + o, 16)]

    def _fire_gather(j, b):
        pltpu.async_copy(x_hbm.at[srcf.at[pl.ds(j * CHUNK, CHUNK)]],
                         bufs[b], gsems[b])

    def _wait_gather(j, b):
        pltpu.make_async_copy(x_hbm.at[srcf.at[pl.ds(j * CHUNK, CHUNK)]],
                              bufs[b], gsems[b]).wait()

    def _fire_scatter(b):
        pltpu.async_copy(bufs[b], acc.at[dvs[b]], ssems[b], add=True)

    def _wait_scatter(b):
        pltpu.make_async_copy(bufs[b], acc.at[dvs[b]], ssems[b]).wait()

    for b in range(NBUF):
        _copy_dst_idx(b, dvs[b])
        _fire_gather(b, b)

    def _group(g, carry):
        base = g * NBUF
        for b in range(NBUF):
            _wait_gather(base + b, b)
            _fire_scatter(b)
        for b in range(NBUF):
            _wait_scatter(b)
            _copy_dst_idx(base + NBUF + b, dvs[b])
            _fire_gather(base + NBUF + b, b)
        return carry

    lax.fori_loop(0, NGROUPS - 1, _group, 0)

    base = (NGROUPS - 1) * NBUF
    for b in range(NBUF):
        _wait_gather(base + b, b)
        _fire_scatter(b)
    for b in range(NBUF):
        _wait_scatter(b)
    plsc.subcore_barrier()

    pltpu.sync_copy(acc.at[pl.ds(sid * RPT, RPT)],
                    out_hbm.at[cid].at[pl.ds(sid * RPT, RPT)])


@functools.cache
def _make_segsum():
    mesh = plsc.VectorSubcoreMesh(core_axis_name="c", subcore_axis_name="s")
    return pl.kernel(
        _segsum_body,
        out_type=jax.ShapeDtypeStruct((NC, NP, D), jnp.float32),
        mesh=mesh,
        scratch_types=[
            pltpu.VMEM((EPT,), jnp.int32),
            pltpu.VMEM((EPT,), jnp.int32),
            pltpu.VMEM_SHARED((NP, D), jnp.float32),
            *[pltpu.VMEM((CHUNK, D), jnp.float32) for _ in range(NBUF)],
            *[pltpu.VMEM((CHUNK,), jnp.int32) for _ in range(NBUF)],
            *[pltpu.SemaphoreType.DMA for _ in range(2 * NBUF)],
        ],
    )


_BLK = 5000


_HI = jax.lax.Precision.HIGHEST


def _rnd(v):
    return v.astype(jnp.bfloat16).astype(jnp.float32)


def _dot_t(a, b):
    return lax.dot_general(a, b, (((1,), (1,)), ((), ())),
                           preferred_element_type=jnp.float32, precision=_HI)


def _root_body(x_ref, wr_ref, o_ref):
    o_ref[...] = _dot_t(x_ref[...], _rnd(wr_ref[...]))


def _combine_body(z_ref, s_ref, we_ref, o_ref):
    s = s_ref[0] + s_ref[1]
    z = z_ref[...] + _dot_t(s, _rnd(we_ref[...]))
    o_ref[...] = _rnd(jnp.maximum(z, 0.0))


def _final_body(z_ref, s_ref, we_ref, wl_ref, o_ref):
    s = s_ref[0] + s_ref[1]
    z = z_ref[...] + _dot_t(s, _rnd(we_ref[...]))
    x2 = _rnd(jnp.maximum(z, 0.0))
    o_ref[...] = _dot_t(x2, _rnd(wl_ref[...]))


_row_spec = pl.BlockSpec((_BLK, D), lambda i: (i, 0))
_s_spec = pl.BlockSpec((NC, _BLK, D), lambda i: (0, i, 0))
_w_spec = pl.BlockSpec((D, D), lambda i: (0, 0))

_root = pl.pallas_call(
    _root_body,
    grid=(N // _BLK,),
    in_specs=[_row_spec, _w_spec],
    out_specs=_row_spec,
    out_shape=jax.ShapeDtypeStruct((N, D), jnp.float32),
)

_combine = pl.pallas_call(
    _combine_body,
    grid=(N // _BLK,),
    in_specs=[_row_spec, _s_spec, _w_spec],
    out_specs=_row_spec,
    out_shape=jax.ShapeDtypeStruct((N, D), jnp.float32),
)

_final = pl.pallas_call(
    _final_body,
    grid=(N // _BLK,),
    in_specs=[_row_spec, _s_spec, _w_spec,
              pl.BlockSpec((1, D), lambda i: (0, 0))],
    out_specs=pl.BlockSpec((_BLK, 1), lambda i: (i, 0)),
    out_shape=jax.ShapeDtypeStruct((N, 1), jnp.float32),
)


def kernel(x, edge_index, batch, W_rel, W_root, W_lin):
    src = edge_index[0].astype(jnp.int32)
    dst = edge_index[1].astype(jnp.int32)
    xb = x.astype(jnp.bfloat16).astype(jnp.float32)

    segsum = _make_segsum()
    s = segsum(xb, src, dst)
    z1 = _root(xb, W_root)
    x1 = _combine(z1, s, W_rel)
    s = segsum(x1, src, dst)
    z2 = _root(x1, W_root)
    return _final(z2, s, W_rel, W_lin)


# speedup vs baseline: 1.0153x; 1.0153x over previous
import functools

import jax
import jax.numpy as jnp
from jax import lax
from jax.experimental import pallas as pl
from jax.experimental.pallas import tpu as pltpu
from jax.experimental.pallas import tpu_sc as plsc

N = 10000
D = 128
E = 320000
NC = 2
NS = 16
NW = NC * NS

EPT = E // NW
CHUNK = 40
CHUNKS = EPT // CHUNK
NP = 10240
RPT = NP // NS
ZCOPIES = RPT // CHUNK

NBUF = 5
NGROUPS = CHUNKS // NBUF


def _segsum_body(x_hbm, src_hbm, dst_hbm, out_hbm, srcf, dstf, acc, *rest):
    bufs = rest[0:NBUF]
    dvs = rest[NBUF:2 * NBUF]
    gsems = rest[2 * NBUF:3 * NBUF]
    ssems = rest[3 * NBUF:4 * NBUF]
    cid = lax.axis_index("c")
    sid = lax.axis_index("s")
    wid = cid * NS + sid

    pltpu.async_copy(src_hbm.at[pl.ds(wid * EPT, EPT)], srcf, gsems[0])
    pltpu.async_copy(dst_hbm.at[pl.ds(wid * EPT, EPT)], dstf, gsems[1])

    zv = jnp.zeros((16,), jnp.float32)

    def _zrow(i, carry):
        for j in range(D // 16):
            bufs[0][i, pl.ds(j * 16, 16)] = zv
        return carry

    lax.fori_loop(0, CHUNK, _zrow, 0)
    for t in range(ZCOPIES):
        pltpu.async_copy(bufs[0], acc.at[pl.ds(sid * RPT + t * CHUNK, CHUNK)],
                         ssems[t % NBUF])
    pltpu.make_async_copy(src_hbm.at[pl.ds(wid * EPT, EPT)], srcf, gsems[0]).wait()
    pltpu.make_async_copy(dst_hbm.at[pl.ds(wid * EPT, EPT)], dstf, gsems[1]).wait()
    for t in range(ZCOPIES):
        pltpu.make_async_copy(bufs[0], acc.at[pl.ds(sid * RPT + t * CHUNK, CHUNK)],
                              ssems[t % NBUF]).wait()
    plsc.subcore_barrier()

    def _copy_dst_idx(j, dv):
        for k in range(CHUNK // 16):
            dv[pl.ds(k * 16, 16)] = dstf[pl.ds(j * CHUNK + k * 16, 16)]
        if CHUNK % 16:
            o = CHUNK - 16
            dv[pl.ds(o, 16)] = dstf[pl.ds(j * CHUNK + o, 16)]

    def _fire_gather(j, b):
        pltpu.async_copy(x_hbm.at[srcf.at[pl.ds(j * CHUNK, CHUNK)]],
                         bufs[b], gsems[b])

    def _wait_gather(j, b):
        pltpu.make_async_copy(x_hbm.at[srcf.at[pl.ds(j * CHUNK, CHUNK)]],
                              bufs[b], gsems[b]).wait()

    def _fire_scatter(b):
        pltpu.async_copy(bufs[b], acc.at[dvs[b]], ssems[b], add=True)

    def _wait_scatter(b):
        pltpu.make_async_copy(bufs[b], acc.at[dvs[b]], ssems[b]).wait()

    for b in range(NBUF):
        _copy_dst_idx(b, dvs[b])
        _fire_gather(b, b)

    def _group(g, carry):
        base = g * NBUF
        for b in range(NBUF):
            _wait_gather(base + b, b)
            _fire_scatter(b)
        for b in range(NBUF):
            _wait_scatter(b)
            _copy_dst_idx(base + NBUF + b, dvs[b])
            _fire_gather(base + NBUF + b, b)
        return carry

    lax.fori_loop(0, NGROUPS - 1, _group, 0)

    base = (NGROUPS - 1) * NBUF
    for b in range(NBUF):
        _wait_gather(base + b, b)
        _fire_scatter(b)
    for b in range(NBUF):
        _wait_scatter(b)
    plsc.subcore_barrier()

    pltpu.sync_copy(acc.at[pl.ds(sid * RPT, RPT)],
                    out_hbm.at[cid].at[pl.ds(sid * RPT, RPT)])


@functools.cache
def _make_segsum():
    mesh = plsc.VectorSubcoreMesh(core_axis_name="c", subcore_axis_name="s")
    return pl.kernel(
        _segsum_body,
        out_type=jax.ShapeDtypeStruct((NC, NP, D), jnp.float32),
        mesh=mesh,
        scratch_types=[
            pltpu.VMEM((EPT,), jnp.int32),
            pltpu.VMEM((EPT,), jnp.int32),
            pltpu.VMEM_SHARED((NP, D), jnp.float32),
            *[pltpu.VMEM((CHUNK, D), jnp.float32) for _ in range(NBUF)],
            *[pltpu.VMEM((CHUNK,), jnp.int32) for _ in range(NBUF)],
            *[pltpu.SemaphoreType.DMA for _ in range(2 * NBUF)],
        ],
    )


_BLK = 2000


_HI = jax.lax.Precision.HIGHEST


def _rnd(v):
    return v.astype(jnp.bfloat16).astype(jnp.float32)


def _dot_t(a, b):
    return lax.dot_general(a, b, (((1,), (1,)), ((), ())),
                           preferred_element_type=jnp.float32, precision=_HI)


def _root_body(x_ref, wr_ref, o_ref):
    o_ref[...] = _dot_t(x_ref[...], _rnd(wr_ref[...]))


def _combine_body(z_ref, s_ref, we_ref, o_ref):
    s = s_ref[0] + s_ref[1]
    z = z_ref[...] + _dot_t(s, _rnd(we_ref[...]))
    o_ref[...] = _rnd(jnp.maximum(z, 0.0))


def _final_body(z_ref, s_ref, we_ref, wl_ref, o_ref):
    s = s_ref[0] + s_ref[1]
    z = z_ref[...] + _dot_t(s, _rnd(we_ref[...]))
    x2 = _rnd(jnp.maximum(z, 0.0))
    o_ref[...] = _dot_t(x2, _rnd(wl_ref[...]))


_row_spec = pl.BlockSpec((_BLK, D), lambda i: (i, 0))
_s_spec = pl.BlockSpec((NC, _BLK, D), lambda i: (0, i, 0))
_w_spec = pl.BlockSpec((D, D), lambda i: (0, 0))

_root = pl.pallas_call(
    _root_body,
    grid=(N // _BLK,),
    in_specs=[_row_spec, _w_spec],
    out_specs=_row_spec,
    out_shape=jax.ShapeDtypeStruct((N, D), jnp.float32),
)

_combine = pl.pallas_call(
    _combine_body,
    grid=(N // _BLK,),
    in_specs=[_row_spec, _s_spec, _w_spec],
    out_specs=_row_spec,
    out_shape=jax.ShapeDtypeStruct((N, D), jnp.float32),
)

_final = pl.pallas_call(
    _final_body,
    grid=(N // _BLK,),
    in_specs=[_row_spec, _s_spec, _w_spec,
              pl.BlockSpec((1, D), lambda i: (0, 0))],
    out_specs=pl.BlockSpec((_BLK, 1), lambda i: (i, 0)),
    out_shape=jax.ShapeDtypeStruct((N, 1), jnp.float32),
)


def kernel(x, edge_index, batch, W_rel, W_root, W_lin):
    src = edge_index[0].astype(jnp.int32)
    dst = edge_index[1].astype(jnp.int32)
    xb = x.astype(jnp.bfloat16).astype(jnp.float32)

    segsum = _make_segsum()
    s = segsum(xb, src, dst)
    z1 = _root(xb, W_root)
    x1 = _combine(z1, s, W_rel)
    s = segsum(x1, src, dst)
    z2 = _root(x1, W_root)
    return _final(z2, s, W_rel, W_lin)
